# baseline (speedup 1.0000x reference)
import functools

import jax
import jax.numpy as jnp
from jax import lax
from jax.experimental import pallas as pl
from jax.experimental.pallas import tpu as pltpu
from jax.experimental.pallas import tpu_sc as plsc

_NC = 2
_NS = 16
_NW = _NC * _NS
_L = 16


_GATHER_DNUMS = lax.GatherDimensionNumbers(
    offset_dims=(), collapsed_slice_dims=(0,), start_index_map=(0,)
)


def _permute16(x, p2d):
  return lax.gather(
      x, p2d, _GATHER_DNUMS, slice_sizes=(1,),
      mode=lax.GatherScatterMode.PROMISE_IN_BOUNDS,
  )


def _rsqrt16(x):
  xi = lax.bitcast_convert_type(x, jnp.int32)
  yi = jnp.int32(0x5F3759DF) - lax.shift_right_arithmetic(
      xi, jnp.full((_L,), 1, jnp.int32))
  y = lax.bitcast_convert_type(yi, jnp.float32)
  for _ in range(3):
    y = y * (1.5 - 0.5 * x * y * y)
  return y


def kernel(idx, emb_weight):
  (batch,) = idx.shape
  n_items, dim = emb_weight.shape
  assert dim == 64 and batch % _NW == 0
  b_per_w = batch // _NW
  idx = idx.astype(jnp.int32)

  mesh = plsc.VectorSubcoreMesh(core_axis_name="c", subcore_axis_name="s")

  @functools.partial(
      pl.kernel,
      out_type=jax.ShapeDtypeStruct((batch, dim), jnp.float32),
      mesh=mesh,
      scratch_types=[
          pltpu.VMEM((b_per_w,), jnp.int32),
          pltpu.VMEM((b_per_w, dim), jnp.float32),
          pltpu.SemaphoreType.DMA,
      ],
      compiler_params=pltpu.CompilerParams(use_tc_tiling_on_sc=False),
  )
  def sc_kernel(idx_hbm, tbl_hbm, out_hbm, idx_v, rows_v, sem):
    wid = lax.axis_index("s") * _NC + lax.axis_index("c")
    base = wid * b_per_w
    pltpu.sync_copy(idx_hbm.at[pl.ds(base, b_per_w)], idx_v)
    pltpu.async_copy(tbl_hbm.at[idx_v], rows_v, sem).wait()

    lane = lax.iota(jnp.int32, _L)
    perms = [
        lax.bitwise_xor(lane, jnp.int32(k)).reshape(_L, 1) for k in (8, 4, 2, 1)
    ]

    def row_body(r, carry):
      chunks = [rows_v[r, pl.ds(j * _L, _L)] for j in range(dim // _L)]
      s = chunks[0] * chunks[0]
      for c in chunks[1:]:
        s = s + c * c
      for p in perms:
        s = s + _permute16(s, p)
      s = jnp.maximum(s, jnp.float32(1e-24))
      inv = _rsqrt16(s)
      for j, c in enumerate(chunks):
        rows_v[r, pl.ds(j * _L, _L)] = c * inv
      return carry

    lax.fori_loop(0, b_per_w, row_body, 0)
    pltpu.sync_copy(rows_v, out_hbm.at[pl.ds(base, b_per_w)])

  return sc_kernel(idx, emb_weight)


# speedup vs baseline: 1.0172x; 1.0172x over previous
import functools

import jax
import jax.numpy as jnp
from jax import lax
from jax.experimental import pallas as pl
from jax.experimental.pallas import tpu as pltpu
from jax.experimental.pallas import tpu_sc as plsc

_NC = 2
_NS = 16
_NW = _NC * _NS
_L = 16


_GATHER_DNUMS = lax.GatherDimensionNumbers(
    offset_dims=(), collapsed_slice_dims=(0,), start_index_map=(0,)
)


def _permute16(x, p2d):
  return lax.gather(
      x, p2d, _GATHER_DNUMS, slice_sizes=(1,),
      mode=lax.GatherScatterMode.PROMISE_IN_BOUNDS,
  )


def _rsqrt16(x):
  xi = lax.bitcast_convert_type(x, jnp.int32)
  yi = jnp.int32(0x5F3759DF) - lax.shift_right_arithmetic(
      xi, jnp.full((_L,), 1, jnp.int32))
  y = lax.bitcast_convert_type(yi, jnp.float32)
  for _ in range(3):
    y = y * (1.5 - 0.5 * x * y * y)
  return y


def kernel(idx, emb_weight):
  (batch,) = idx.shape
  n_items, dim = emb_weight.shape
  assert dim == 64 and batch % _NW == 0
  b_per_w = batch // _NW
  idx = idx.astype(jnp.int32)

  mesh = plsc.VectorSubcoreMesh(core_axis_name="c", subcore_axis_name="s")

  @functools.partial(
      pl.kernel,
      out_type=jax.ShapeDtypeStruct((batch, dim), jnp.float32),
      mesh=mesh,
      scratch_types=[
          pltpu.VMEM((b_per_w,), jnp.int32),
          pltpu.VMEM((b_per_w, dim), jnp.float32),
          pltpu.VMEM((_L, _L), jnp.float32),
          pltpu.SemaphoreType.DMA,
      ],
      compiler_params=pltpu.CompilerParams(
          use_tc_tiling_on_sc=False, needs_layout_passes=False),
  )
  def sc_kernel(idx_hbm, tbl_hbm, out_hbm, idx_v, rows_v, part_v, sem):
    wid = lax.axis_index("s") * _NC + lax.axis_index("c")
    base = wid * b_per_w
    pltpu.sync_copy(idx_hbm.at[pl.ds(base, b_per_w)], idx_v)
    pltpu.async_copy(tbl_hbm.at[idx_v], rows_v, sem).wait()

    lane = lax.iota(jnp.int32, _L)
    splats = [jnp.full((_L,), r, jnp.int32) for r in range(_L)]
    splats2d = [s.reshape(_L, 1) for s in splats]
    nchunk = dim // _L

    def group_body(g, carry):
      b0 = g * _L
      for r in range(_L):
        c = [rows_v[b0 + r, pl.ds(j * _L, _L)] for j in range(nchunk)]
        s = c[0] * c[0]
        for cc in c[1:]:
          s = s + cc * cc
        part_v[r, :] = s
      tot = plsc.load_gather(part_v, [lane, splats[0]])
      for l in range(1, _L):
        tot = tot + plsc.load_gather(part_v, [lane, splats[l]])
      inv = _rsqrt16(jnp.maximum(tot, jnp.float32(1e-24)))
      for r in range(_L):
        scale = _permute16(inv, splats2d[r])
        for j in range(nchunk):
          rows_v[b0 + r, pl.ds(j * _L, _L)] = (
              rows_v[b0 + r, pl.ds(j * _L, _L)] * scale)
      return carry

    lax.fori_loop(0, b_per_w // _L, group_body, 0)
    pltpu.sync_copy(rows_v, out_hbm.at[pl.ds(base, b_per_w)])

  return sc_kernel(idx, emb_weight)


# speedup vs baseline: 2.3463x; 2.3067x over previous
import functools

import jax
import jax.numpy as jnp
from jax import lax
from jax.experimental import pallas as pl
from jax.experimental.pallas import tpu as pltpu
from jax.experimental.pallas import tpu_sc as plsc

_NC = 2
_NS = 16
_NW = _NC * _NS
_L = 16
_SUB = 8


def _rsqrt16(x):
  xi = lax.bitcast_convert_type(x, jnp.int32)
  yi = jnp.int32(0x5F3759DF) - lax.shift_right_arithmetic(
      xi, jnp.full((_L,), 1, jnp.int32))
  y = lax.bitcast_convert_type(yi, jnp.float32)
  for _ in range(3):
    y = y * (1.5 - 0.5 * x * y * y)
  return y


def kernel(idx, emb_weight):
  (batch,) = idx.shape
  n_items, dim = emb_weight.shape
  assert dim == 64 and n_items % _SUB == 0 and batch % (_NW * _L) == 0
  b_per_w = batch // _NW
  n_grp = b_per_w // _L
  idx = idx.astype(jnp.int32)
  tbl3 = emb_weight.reshape(n_items // _SUB, _SUB, dim)

  mesh = plsc.VectorSubcoreMesh(core_axis_name="c", subcore_axis_name="s")

  @functools.partial(
      pl.kernel,
      out_type=jax.ShapeDtypeStruct((batch, dim), jnp.float32),
      mesh=mesh,
      scratch_types=[
          pltpu.VMEM((b_per_w,), jnp.int32),
          pltpu.VMEM((_L, _SUB, dim), jnp.float32),
          pltpu.VMEM((_L, _SUB, dim), jnp.float32),
          pltpu.VMEM((_L, dim), jnp.float32),
          pltpu.VMEM((_L, dim), jnp.float32),
          pltpu.VMEM((_L, _L), jnp.float32),
          pltpu.SemaphoreType.DMA,
          pltpu.SemaphoreType.DMA,
          pltpu.SemaphoreType.DMA,
          pltpu.SemaphoreType.DMA,
      ],
      compiler_params=pltpu.CompilerParams(needs_layout_passes=False),
  )
  def sc_kernel(idx_hbm, tbl_hbm, out_hbm, idx_v, blk0, blk1, stg0, stg1,
                part_v, gsem0, gsem1, osem0, osem1):
    wid = lax.axis_index("s") * _NC + lax.axis_index("c")
    base = wid * b_per_w
    pltpu.sync_copy(idx_hbm.at[pl.ds(base, b_per_w)], idx_v)

    blks = (blk0, blk1)
    stgs = (stg0, stg1)
    gsems = (gsem0, gsem1)
    osems = (osem0, osem1)
    lane = lax.iota(jnp.int32, _L)
    splats = [jnp.full((_L,), l, jnp.int32) for l in range(_L)]
    three = jnp.full((_L,), 3, jnp.int32)

    def fetch(g, slot):
      gv = idx_v[pl.ds(g * _L, _L)]
      bv = lax.shift_right_logical(gv, three)
      for k in range(_L):
        pltpu.async_copy(tbl_hbm.at[bv[k]], blks[slot].at[k], gsems[slot])

    fetch(0, 0)
    fetch(1, 1)

    def pair_body(t, carry):
      for slot in range(2):
        g = 2 * t + slot
        blk, stg = blks[slot], stgs[slot]
        for k in range(_L):
          pltpu.make_async_copy(
              tbl_hbm.at[0], blk.at[0], gsems[slot]).wait()
        gv = idx_v[pl.ds(g * _L, _L)]

        for r in range(_L):
          srow = lax.bitwise_and(gv[r], jnp.int32(_SUB - 1))
          s = jnp.zeros((_L,), jnp.float32)
          for q in range(dim // _L):
            v = blk[r, srow, pl.ds(q * _L, _L)]
            s = s + v * v
          part_v[r, :] = s
        tot = plsc.load_gather(part_v, [lane, splats[0]])
        for l in range(1, _L):
          tot = tot + plsc.load_gather(part_v, [lane, splats[l]])
        inv = _rsqrt16(jnp.maximum(tot, jnp.float32(1e-24)))

        @pl.when(g >= 2)
        def _():
          pltpu.make_async_copy(
              stg, out_hbm.at[pl.ds(0, _L)], osems[slot]).wait()

        for r in range(_L):
          srow = lax.bitwise_and(gv[r], jnp.int32(_SUB - 1))
          iv = jnp.full((_L,), inv[r], jnp.float32)
          for q in range(dim // _L):
            sl = pl.ds(q * _L, _L)
            stg[r, sl] = blk[r, srow, sl] * iv
        pltpu.async_copy(
            stg, out_hbm.at[pl.ds(base + g * _L, _L)], osems[slot])

        @pl.when(g + 2 < n_grp)
        def _():
          fetch(g + 2, slot)
      return carry

    lax.fori_loop(0, n_grp // 2, pair_body, 0)
    for slot in range(2):
      pltpu.make_async_copy(
          stgs[slot], out_hbm.at[pl.ds(0, _L)], osems[slot]).wait()

  return sc_kernel(idx, tbl3)
